# Initial kernel scaffold; baseline (speedup 1.0000x reference)
import jax
import jax.numpy as jnp
from jax.experimental import pallas as pl


def kernel(x, edge_index, W1_neigh, b1, W2_self, W2_neigh, b2, W_pred, b_pred):
    raise NotImplementedError("write your pallas kernel here")


# speedup vs baseline: 4.0954x; 4.0954x over previous
import functools

import jax
import jax.numpy as jnp
from jax import lax
from jax.experimental import pallas as pl
from jax.experimental.pallas import tpu as pltpu
from jax.experimental.pallas import tpu_sc as plsc

N = 10000
NP = 10240
E = 320000
D = 128
DEGW = 8

NC, NS, L = 2, 16, 16
NW = NC * NS
EPW = E // NW
CH = 80
NCHUNK = EPW // CH
RPT = NP // NS

ROWS_TC = 2048


def _sc_mesh():
    return plsc.VectorSubcoreMesh(
        core_axis_name="c", subcore_axis_name="s",
        num_cores=NC, num_subcores=NS)


DH = D // 2


def _seg_sum_body(with_deg, *refs):
    if with_deg:
        (table_hbm, srca_hbm, srcb_hbm, dst_hbm, zf_hbm, zd_hbm, ones_hbm,
         outa_hbm, outb_hbm, deg_hbm,
         sidx, didx, rows, ones_v, acc_sh, deg_sh, sem) = refs
    else:
        (table_hbm, srca_hbm, srcb_hbm, dst_hbm, zf_hbm,
         outa_hbm, outb_hbm,
         sidx, didx, rows, acc_sh, sem) = refs

    c = lax.axis_index("c")
    s = lax.axis_index("s")
    wid = s * NC + c
    r0 = s * RPT
    base = wid * EPW

    for half in range(2):
        src_hbm = srca_hbm if half == 0 else srcb_hbm
        out_hbm = outa_hbm if half == 0 else outb_hbm
        first = with_deg and half == 0

        for j in range(RPT // CH):
            q = r0 + j * CH
            pltpu.sync_copy(zf_hbm.at[pl.ds(q, CH)], rows)
            pltpu.sync_copy(rows, acc_sh.at[pl.ds(q, CH)])
            if first:
                pltpu.sync_copy(zd_hbm.at[pl.ds(q, CH)], ones_v)
                pltpu.sync_copy(ones_v, deg_sh.at[pl.ds(q, CH)])
        if first:
            pltpu.sync_copy(ones_hbm, ones_v)
        plsc.subcore_barrier()

        def step(k, carry):
            off = base + k * CH
            pltpu.sync_copy(src_hbm.at[pl.ds(off, CH)], sidx)
            pltpu.sync_copy(dst_hbm.at[pl.ds(off, CH)], didx)
            pltpu.async_copy(table_hbm.at[sidx], rows, sem).wait()
            pltpu.sync_copy(rows, acc_sh.at[didx], add=True)
            if first:
                pltpu.sync_copy(ones_v, deg_sh.at[didx], add=True)
            return carry

        lax.fori_loop(0, NCHUNK, step, 0)
        plsc.subcore_barrier()

        for j in range(RPT // CH):
            q = r0 + j * CH
            pltpu.sync_copy(acc_sh.at[pl.ds(q, CH)], rows)
            pltpu.sync_copy(rows, out_hbm.at[c, pl.ds(q, CH)])
            if first:
                pltpu.sync_copy(deg_sh.at[pl.ds(q, CH)], ones_v)
                pltpu.sync_copy(ones_v, deg_hbm.at[c, pl.ds(q, CH)])


def _seg_sum_deg(table2, srca, srcb, dst, zf, zd, ones8):
    scratch = [
        pltpu.VMEM((CH,), jnp.int32),
        pltpu.VMEM((CH,), jnp.int32),
        pltpu.VMEM((CH, DH), jnp.float32),
        pltpu.VMEM((CH, DEGW), jnp.float32),
        pltpu.VMEM_SHARED((NP, DH), jnp.float32),
        pltpu.VMEM_SHARED((NP, DEGW), jnp.float32),
        pltpu.SemaphoreType.DMA,
    ]
    out_type = (jax.ShapeDtypeStruct((NC, NP, DH), jnp.float32),
                jax.ShapeDtypeStruct((NC, NP, DH), jnp.float32),
                jax.ShapeDtypeStruct((NC, NP, DEGW), jnp.float32))
    return pl.kernel(functools.partial(_seg_sum_body, True), out_type,
                     mesh=_sc_mesh(), scratch_types=scratch,
                     compiler_params=pltpu.CompilerParams(
                         use_tc_tiling_on_sc=False),
                     name="sc_seg_sum_deg")(table2, srca, srcb, dst, zf, zd, ones8)


def _seg_sum(table2, srca, srcb, dst, zf):
    scratch = [
        pltpu.VMEM((CH,), jnp.int32),
        pltpu.VMEM((CH,), jnp.int32),
        pltpu.VMEM((CH, DH), jnp.float32),
        pltpu.VMEM_SHARED((NP, DH), jnp.float32),
        pltpu.SemaphoreType.DMA,
    ]
    out_type = (jax.ShapeDtypeStruct((NC, NP, DH), jnp.float32),
                jax.ShapeDtypeStruct((NC, NP, DH), jnp.float32))
    return pl.kernel(functools.partial(_seg_sum_body, False), out_type,
                     mesh=_sc_mesh(), scratch_types=scratch,
                     compiler_params=pltpu.CompilerParams(
                         use_tc_tiling_on_sc=False),
                     name="sc_seg_sum")(table2, srca, srcb, dst, zf)


def _edge_score_body(ac_hbm, src_hbm, dst_hbm, out_hbm,
                     ac_v, sidx_v, didx_v, out_v):
    c = lax.axis_index("c")
    s = lax.axis_index("s")
    wid = s * NC + c
    base = wid * EPW

    pltpu.sync_copy(ac_hbm, ac_v)
    pltpu.sync_copy(src_hbm.at[pl.ds(base, EPW)], sidx_v)
    pltpu.sync_copy(dst_hbm.at[pl.ds(base, EPW)], didx_v)

    def step(i, carry):
        o = i * L
        sv = sidx_v[pl.ds(o, L)]
        dv = didx_v[pl.ds(o, L)]
        a = plsc.load_gather(ac_v, [sv * 2])
        b = plsc.load_gather(ac_v, [dv * 2 + 1])
        out_v[pl.ds(o, L)] = a + b
        return carry

    lax.fori_loop(0, EPW // L, step, 0)
    pltpu.sync_copy(out_v, out_hbm.at[pl.ds(base, EPW)])


def _edge_score(ac, src, dst):
    scratch = [
        pltpu.VMEM((NP * 2,), jnp.float32),
        pltpu.VMEM((EPW,), jnp.int32),
        pltpu.VMEM((EPW,), jnp.int32),
        pltpu.VMEM((EPW,), jnp.float32),
    ]
    out_type = jax.ShapeDtypeStruct((E,), jnp.float32)
    return pl.kernel(_edge_score_body, out_type,
                     mesh=_sc_mesh(), scratch_types=scratch,
                     compiler_params=pltpu.CompilerParams(
                         needs_layout_passes=False),
                     name="sc_edge_score")(ac, src, dst)


def _conv1_body(x_b, agga_b, aggb_b, deg_b, w_b, b_b, out_b):
    deg = deg_b[0, :, 0:1] + deg_b[1, :, 0:1]
    agg = jnp.concatenate([agga_b[0] + agga_b[1], aggb_b[0] + aggb_b[1]],
                          axis=1)
    hn = (agg + x_b[...]) / (deg + 1.0)
    acc = jnp.dot(hn, w_b[...], preferred_element_type=jnp.float32)
    out_b[...] = jnp.maximum(acc + b_b[...], 0.0)


def _conv1(x, agg1a, agg1b, degp, W1, b1):
    grid = (NP // ROWS_TC,)
    return pl.pallas_call(
        _conv1_body,
        grid=grid,
        in_specs=[
            pl.BlockSpec((ROWS_TC, D), lambda i: (i, 0)),
            pl.BlockSpec((NC, ROWS_TC, DH), lambda i: (0, i, 0)),
            pl.BlockSpec((NC, ROWS_TC, DH), lambda i: (0, i, 0)),
            pl.BlockSpec((NC, ROWS_TC, DEGW), lambda i: (0, i, 0)),
            pl.BlockSpec((D, D), lambda i: (0, 0)),
            pl.BlockSpec((1, D), lambda i: (0, 0)),
        ],
        out_specs=pl.BlockSpec((ROWS_TC, D), lambda i: (i, 0)),
        out_shape=jax.ShapeDtypeStruct((NP, D), jnp.float32),
    )(x, agg1a, agg1b, degp, W1, b1.reshape(1, D))


def _conv2_body(h_b, agga_b, aggb_b, deg_b, ws_b, wn_b, b_b, wp_b, bp_b, out_b):
    deg = deg_b[0, :, 0:1] + deg_b[1, :, 0:1]
    degc = jnp.maximum(deg, 1.0)
    agg = jnp.concatenate([agga_b[0] + agga_b[1], aggb_b[0] + aggb_b[1]],
                          axis=1)
    hn = agg / degc
    h2 = (jnp.dot(h_b[...], ws_b[...], preferred_element_type=jnp.float32)
          + jnp.dot(hn, wn_b[...], preferred_element_type=jnp.float32)
          + b_b[...])
    out_b[...] = jnp.dot(h2, wp_b[...], preferred_element_type=jnp.float32) + bp_b[...]


def _conv2(h, agg2a, agg2b, degp, W2s, W2n, b2, Wp2, bp2):
    grid = (NP // ROWS_TC,)
    return pl.pallas_call(
        _conv2_body,
        grid=grid,
        in_specs=[
            pl.BlockSpec((ROWS_TC, D), lambda i: (i, 0)),
            pl.BlockSpec((NC, ROWS_TC, DH), lambda i: (0, i, 0)),
            pl.BlockSpec((NC, ROWS_TC, DH), lambda i: (0, i, 0)),
            pl.BlockSpec((NC, ROWS_TC, DEGW), lambda i: (0, i, 0)),
            pl.BlockSpec((D, D), lambda i: (0, 0)),
            pl.BlockSpec((D, D), lambda i: (0, 0)),
            pl.BlockSpec((1, D), lambda i: (0, 0)),
            pl.BlockSpec((D, 2), lambda i: (0, 0)),
            pl.BlockSpec((1, 2), lambda i: (0, 0)),
        ],
        out_specs=pl.BlockSpec((ROWS_TC, 2), lambda i: (i, 0)),
        out_shape=jax.ShapeDtypeStruct((NP, 2), jnp.float32),
    )(h, agg2a, agg2b, degp, W2s, W2n, b2.reshape(1, D), Wp2, bp2)


def kernel(x, edge_index, W1_neigh, b1, W2_self, W2_neigh, b2, W_pred, b_pred):
    src = edge_index[0]
    dst = edge_index[1]
    srca = src * 2
    srcb = src * 2 + 1
    xp = jnp.pad(x, ((0, NP - N), (0, 0)))
    zf = jnp.zeros((NP, DH), jnp.float32)
    zd = jnp.zeros((NP, DEGW), jnp.float32)
    ones8 = jnp.ones((CH, DEGW), jnp.float32)

    agg1a, agg1b, degp = _seg_sum_deg(
        xp.reshape(NP * 2, DH), srca, srcb, dst, zf, zd, ones8)
    h = _conv1(xp, agg1a, agg1b, degp, W1_neigh, b1)
    agg2a, agg2b = _seg_sum(h.reshape(NP * 2, DH), srca, srcb, dst, zf)

    Wp2 = W_pred[:, 0].reshape(2, D).T
    bp2 = jnp.concatenate([b_pred, jnp.zeros((1,), jnp.float32)]).reshape(1, 2)
    ac = _conv2(h, agg2a, agg2b, degp, W2_self, W2_neigh, b2, Wp2, bp2)

    score = _edge_score(ac.reshape(NP * 2), src, dst)
    return score.reshape(E, 1)


# speedup vs baseline: 6.1678x; 1.5060x over previous
import functools

import jax
import jax.numpy as jnp
from jax import lax
from jax.experimental import pallas as pl
from jax.experimental.pallas import tpu as pltpu
from jax.experimental.pallas import tpu_sc as plsc

N = 10000
NP = 10240
E = 320000
D = 128
DEGW = 8

NC, NS, L = 2, 16, 16
NW = NC * NS
EPW = E // NW
CH = 80
NCHUNK = EPW // CH
RPT = NP // NS

ROWS_TC = 2048


def _sc_mesh():
    return plsc.VectorSubcoreMesh(
        core_axis_name="c", subcore_axis_name="s",
        num_cores=NC, num_subcores=NS)


DH = D // 2


def _seg_sum_body(with_deg, *refs):
    if with_deg:
        (table_hbm, srca_hbm, srcb_hbm, dst_hbm, zf_hbm, zd_hbm, ones_hbm,
         outa_hbm, outb_hbm, deg_hbm,
         sidx0, sidx1, didx0, didx1, rows0, rows1, ones_v,
         acc_sh, deg_sh, sem0, sem1) = refs
    else:
        (table_hbm, srca_hbm, srcb_hbm, dst_hbm, zf_hbm,
         outa_hbm, outb_hbm,
         sidx0, sidx1, didx0, didx1, rows0, rows1,
         acc_sh, sem0, sem1) = refs

    c = lax.axis_index("c")
    s = lax.axis_index("s")
    wid = s * NC + c
    r0 = s * RPT
    base = wid * EPW
    sidx = (sidx0, sidx1)
    didx = (didx0, didx1)
    rows = (rows0, rows1)
    sems = (sem0, sem1)

    for half in range(2):
        src_hbm = srca_hbm if half == 0 else srcb_hbm
        out_hbm = outa_hbm if half == 0 else outb_hbm
        first = with_deg and half == 0

        for j in range(RPT // CH):
            q = r0 + j * CH
            pltpu.sync_copy(zf_hbm.at[pl.ds(q, CH)], rows0)
            pltpu.sync_copy(rows0, acc_sh.at[pl.ds(q, CH)])
            if first:
                pltpu.sync_copy(zd_hbm.at[pl.ds(q, CH)], ones_v)
                pltpu.sync_copy(ones_v, deg_sh.at[pl.ds(q, CH)])
        if first:
            pltpu.sync_copy(ones_hbm, ones_v)
        plsc.subcore_barrier()

        def fetch(k, b):
            off = base + k * CH
            pltpu.sync_copy(src_hbm.at[pl.ds(off, CH)], sidx[b])
            pltpu.sync_copy(dst_hbm.at[pl.ds(off, CH)], didx[b])
            pltpu.async_copy(table_hbm.at[sidx[b]], rows[b], sems[b])

        def drain(b):
            pltpu.make_async_copy(table_hbm.at[sidx[b]], rows[b],
                                  sems[b]).wait()
            pltpu.sync_copy(rows[b], acc_sh.at[didx[b]], add=True)
            if first:
                pltpu.sync_copy(ones_v, deg_sh.at[didx[b]], add=True)

        fetch(0, 0)

        def step(j, carry):
            k = j * 2
            fetch(k + 1, 1)
            drain(0)
            fetch(k + 2, 0)
            drain(1)
            return carry

        lax.fori_loop(0, (NCHUNK - 1) // 2, step, 0)
        drain(0)
        plsc.subcore_barrier()

        for j in range(RPT // CH):
            q = r0 + j * CH
            pltpu.sync_copy(acc_sh.at[pl.ds(q, CH)], rows0)
            pltpu.sync_copy(rows0, out_hbm.at[c, pl.ds(q, CH)])
            if first:
                pltpu.sync_copy(deg_sh.at[pl.ds(q, CH)], ones_v)
                pltpu.sync_copy(ones_v, deg_hbm.at[c, pl.ds(q, CH)])


def _seg_sum_deg(table2, srca, srcb, dst, zf, zd, ones8):
    scratch = [
        pltpu.VMEM((CH,), jnp.int32),
        pltpu.VMEM((CH,), jnp.int32),
        pltpu.VMEM((CH,), jnp.int32),
        pltpu.VMEM((CH,), jnp.int32),
        pltpu.VMEM((CH, DH), jnp.float32),
        pltpu.VMEM((CH, DH), jnp.float32),
        pltpu.VMEM((CH, DEGW), jnp.float32),
        pltpu.VMEM_SHARED((NP, DH), jnp.float32),
        pltpu.VMEM_SHARED((NP, DEGW), jnp.float32),
        pltpu.SemaphoreType.DMA,
        pltpu.SemaphoreType.DMA,
    ]
    out_type = (jax.ShapeDtypeStruct((NC, NP, DH), jnp.float32),
                jax.ShapeDtypeStruct((NC, NP, DH), jnp.float32),
                jax.ShapeDtypeStruct((NC, NP, DEGW), jnp.float32))
    return pl.kernel(functools.partial(_seg_sum_body, True), out_type,
                     mesh=_sc_mesh(), scratch_types=scratch,
                     compiler_params=pltpu.CompilerParams(
                         use_tc_tiling_on_sc=False),
                     name="sc_seg_sum_deg")(table2, srca, srcb, dst, zf, zd, ones8)


def _seg_sum(table2, srca, srcb, dst, zf):
    scratch = [
        pltpu.VMEM((CH,), jnp.int32),
        pltpu.VMEM((CH,), jnp.int32),
        pltpu.VMEM((CH,), jnp.int32),
        pltpu.VMEM((CH,), jnp.int32),
        pltpu.VMEM((CH, DH), jnp.float32),
        pltpu.VMEM((CH, DH), jnp.float32),
        pltpu.VMEM_SHARED((NP, DH), jnp.float32),
        pltpu.SemaphoreType.DMA,
        pltpu.SemaphoreType.DMA,
    ]
    out_type = (jax.ShapeDtypeStruct((NC, NP, DH), jnp.float32),
                jax.ShapeDtypeStruct((NC, NP, DH), jnp.float32))
    return pl.kernel(functools.partial(_seg_sum_body, False), out_type,
                     mesh=_sc_mesh(), scratch_types=scratch,
                     compiler_params=pltpu.CompilerParams(
                         use_tc_tiling_on_sc=False),
                     name="sc_seg_sum")(table2, srca, srcb, dst, zf)


def _edge_score_body(ac_hbm, src_hbm, dst_hbm, out_hbm,
                     ac_v, sidx_v, didx_v, out_v):
    c = lax.axis_index("c")
    s = lax.axis_index("s")
    wid = s * NC + c
    base = wid * EPW

    pltpu.sync_copy(ac_hbm, ac_v)
    pltpu.sync_copy(src_hbm.at[pl.ds(base, EPW)], sidx_v)
    pltpu.sync_copy(dst_hbm.at[pl.ds(base, EPW)], didx_v)

    def step(i, carry):
        o = i * L
        sv = sidx_v[pl.ds(o, L)]
        dv = didx_v[pl.ds(o, L)]
        a = plsc.load_gather(ac_v, [sv * 2])
        b = plsc.load_gather(ac_v, [dv * 2 + 1])
        out_v[pl.ds(o, L)] = a + b
        return carry

    lax.fori_loop(0, EPW // L, step, 0)
    pltpu.sync_copy(out_v, out_hbm.at[pl.ds(base, EPW)])


def _edge_score(ac, src, dst):
    scratch = [
        pltpu.VMEM((NP * 2,), jnp.float32),
        pltpu.VMEM((EPW,), jnp.int32),
        pltpu.VMEM((EPW,), jnp.int32),
        pltpu.VMEM((EPW,), jnp.float32),
    ]
    out_type = jax.ShapeDtypeStruct((E,), jnp.float32)
    return pl.kernel(_edge_score_body, out_type,
                     mesh=_sc_mesh(), scratch_types=scratch,
                     compiler_params=pltpu.CompilerParams(
                         needs_layout_passes=False),
                     name="sc_edge_score")(ac, src, dst)


def _conv1_body(x_b, agga_b, aggb_b, deg_b, w_b, b_b, out_b):
    deg = deg_b[0, :, 0:1] + deg_b[1, :, 0:1]
    agg = jnp.concatenate([agga_b[0] + agga_b[1], aggb_b[0] + aggb_b[1]],
                          axis=1)
    hn = (agg + x_b[...]) / (deg + 1.0)
    acc = jnp.dot(hn, w_b[...], preferred_element_type=jnp.float32)
    out_b[...] = jnp.maximum(acc + b_b[...], 0.0)


def _conv1(x, agg1a, agg1b, degp, W1, b1):
    grid = (NP // ROWS_TC,)
    return pl.pallas_call(
        _conv1_body,
        grid=grid,
        in_specs=[
            pl.BlockSpec((ROWS_TC, D), lambda i: (i, 0)),
            pl.BlockSpec((NC, ROWS_TC, DH), lambda i: (0, i, 0)),
            pl.BlockSpec((NC, ROWS_TC, DH), lambda i: (0, i, 0)),
            pl.BlockSpec((NC, ROWS_TC, DEGW), lambda i: (0, i, 0)),
            pl.BlockSpec((D, D), lambda i: (0, 0)),
            pl.BlockSpec((1, D), lambda i: (0, 0)),
        ],
        out_specs=pl.BlockSpec((ROWS_TC, D), lambda i: (i, 0)),
        out_shape=jax.ShapeDtypeStruct((NP, D), jnp.float32),
    )(x, agg1a, agg1b, degp, W1, b1.reshape(1, D))


def _conv2_body(h_b, agga_b, aggb_b, deg_b, ws_b, wn_b, b_b, wp_b, bp_b, out_b):
    deg = deg_b[0, :, 0:1] + deg_b[1, :, 0:1]
    degc = jnp.maximum(deg, 1.0)
    agg = jnp.concatenate([agga_b[0] + agga_b[1], aggb_b[0] + aggb_b[1]],
                          axis=1)
    hn = agg / degc
    h2 = (jnp.dot(h_b[...], ws_b[...], preferred_element_type=jnp.float32)
          + jnp.dot(hn, wn_b[...], preferred_element_type=jnp.float32)
          + b_b[...])
    out_b[...] = jnp.dot(h2, wp_b[...], preferred_element_type=jnp.float32) + bp_b[...]


def _conv2(h, agg2a, agg2b, degp, W2s, W2n, b2, Wp2, bp2):
    grid = (NP // ROWS_TC,)
    return pl.pallas_call(
        _conv2_body,
        grid=grid,
        in_specs=[
            pl.BlockSpec((ROWS_TC, D), lambda i: (i, 0)),
            pl.BlockSpec((NC, ROWS_TC, DH), lambda i: (0, i, 0)),
            pl.BlockSpec((NC, ROWS_TC, DH), lambda i: (0, i, 0)),
            pl.BlockSpec((NC, ROWS_TC, DEGW), lambda i: (0, i, 0)),
            pl.BlockSpec((D, D), lambda i: (0, 0)),
            pl.BlockSpec((D, D), lambda i: (0, 0)),
            pl.BlockSpec((1, D), lambda i: (0, 0)),
            pl.BlockSpec((D, 2), lambda i: (0, 0)),
            pl.BlockSpec((1, 2), lambda i: (0, 0)),
        ],
        out_specs=pl.BlockSpec((ROWS_TC, 2), lambda i: (i, 0)),
        out_shape=jax.ShapeDtypeStruct((NP, 2), jnp.float32),
    )(h, agg2a, agg2b, degp, W2s, W2n, b2.reshape(1, D), Wp2, bp2)


def kernel(x, edge_index, W1_neigh, b1, W2_self, W2_neigh, b2, W_pred, b_pred):
    src = edge_index[0]
    dst = edge_index[1]
    srca = src * 2
    srcb = src * 2 + 1
    xp = jnp.pad(x, ((0, NP - N), (0, 0)))
    zf = jnp.zeros((NP, DH), jnp.float32)
    zd = jnp.zeros((NP, DEGW), jnp.float32)
    ones8 = jnp.ones((CH, DEGW), jnp.float32)

    agg1a, agg1b, degp = _seg_sum_deg(
        xp.reshape(NP * 2, DH), srca, srcb, dst, zf, zd, ones8)
    h = _conv1(xp, agg1a, agg1b, degp, W1_neigh, b1)
    agg2a, agg2b = _seg_sum(h.reshape(NP * 2, DH), srca, srcb, dst, zf)

    Wp2 = W_pred[:, 0].reshape(2, D).T
    bp2 = jnp.concatenate([b_pred, jnp.zeros((1,), jnp.float32)]).reshape(1, 2)
    ac = _conv2(h, agg2a, agg2b, degp, W2_self, W2_neigh, b2, Wp2, bp2)

    score = _edge_score(ac.reshape(NP * 2), src, dst)
    return score.reshape(E, 1)


# speedup vs baseline: 10.6153x; 1.7211x over previous
import functools

import jax
import jax.numpy as jnp
from jax import lax
from jax.experimental import pallas as pl
from jax.experimental.pallas import tpu as pltpu
from jax.experimental.pallas import tpu_sc as plsc

N = 10000
NP = 10240
E = 320000
D = 128
DEGW = 8

NC, NS, L = 2, 16, 16
EPC = E // NS
CH = 80
NCHUNK = EPC // CH
RPT = NP // NS

ROWS_TC = 2048

DH = D // 2


def _sc_mesh():
    return plsc.VectorSubcoreMesh(
        core_axis_name="c", subcore_axis_name="s",
        num_cores=NC, num_subcores=NS)


def _seg_sum_body(with_deg, *refs):
    if with_deg:
        (table_hbm, src2_hbm, dst_hbm, zf_hbm, zd_hbm, ones_hbm,
         out_hbm, deg_hbm,
         sidx_all, didx_all, rows0, rows1, ones_v,
         acc_sh, deg_sh, sem0, sem1) = refs
    else:
        (table_hbm, src2_hbm, dst_hbm, zf_hbm,
         out_hbm,
         sidx_all, didx_all, rows0, rows1,
         acc_sh, sem0, sem1) = refs

    c = lax.axis_index("c")
    s = lax.axis_index("s")
    r0 = s * RPT
    base = s * EPC
    rows = (rows0, rows1)
    sems = (sem0, sem1)

    for j in range(RPT // CH):
        q = r0 + j * CH
        pltpu.sync_copy(zf_hbm.at[pl.ds(q, CH)], rows0)
        pltpu.sync_copy(rows0, acc_sh.at[pl.ds(q, CH)])
        if with_deg:
            pltpu.sync_copy(zd_hbm.at[pl.ds(q, CH)], ones_v)
            pltpu.sync_copy(ones_v, deg_sh.at[pl.ds(q, CH)])
    if with_deg:
        pltpu.sync_copy(ones_hbm, ones_v)

    pltpu.sync_copy(src2_hbm.at[c, pl.ds(base, EPC)], sidx_all)
    pltpu.sync_copy(dst_hbm.at[pl.ds(base, EPC)], didx_all)
    plsc.subcore_barrier()

    def fetch(k, b):
        off = k * CH
        pltpu.async_copy(table_hbm.at[sidx_all.at[pl.ds(off, CH)]],
                         rows[b], sems[b])

    def drain(k, b):
        off = k * CH
        didx = didx_all.at[pl.ds(off, CH)]
        pltpu.make_async_copy(table_hbm.at[sidx_all.at[pl.ds(off, CH)]],
                              rows[b], sems[b]).wait()
        pltpu.sync_copy(rows[b], acc_sh.at[didx], add=True)
        if with_deg:
            pltpu.sync_copy(ones_v, deg_sh.at[didx], add=True)

    fetch(0, 0)
    fetch(1, 1)

    def step(j, carry):
        k = j * 2
        drain(k, 0)
        fetch(k + 2, 0)
        drain(k + 1, 1)
        fetch(k + 3, 1)
        return carry

    lax.fori_loop(0, (NCHUNK - 2) // 2, step, 0)
    drain(NCHUNK - 2, 0)
    drain(NCHUNK - 1, 1)
    plsc.subcore_barrier()

    for j in range(RPT // CH):
        q = r0 + j * CH
        pltpu.sync_copy(acc_sh.at[pl.ds(q, CH)], rows0)
        pltpu.sync_copy(rows0, out_hbm.at[c, pl.ds(q, CH)])
        if with_deg:
            pltpu.sync_copy(deg_sh.at[pl.ds(q, CH)], ones_v)
            pltpu.sync_copy(ones_v, deg_hbm.at[c, pl.ds(q, CH)])


def _seg_sum_deg(table2, src2, dst, zf, zd, ones8):
    scratch = [
        pltpu.VMEM((EPC,), jnp.int32),
        pltpu.VMEM((EPC,), jnp.int32),
        pltpu.VMEM((CH, DH), jnp.float32),
        pltpu.VMEM((CH, DH), jnp.float32),
        pltpu.VMEM((CH, DEGW), jnp.float32),
        pltpu.VMEM_SHARED((NP, DH), jnp.float32),
        pltpu.VMEM_SHARED((NP, DEGW), jnp.float32),
        pltpu.SemaphoreType.DMA,
        pltpu.SemaphoreType.DMA,
    ]
    out_type = (jax.ShapeDtypeStruct((NC, NP, DH), jnp.float32),
                jax.ShapeDtypeStruct((NC, NP, DEGW), jnp.float32))
    return pl.kernel(functools.partial(_seg_sum_body, True), out_type,
                     mesh=_sc_mesh(), scratch_types=scratch,
                     compiler_params=pltpu.CompilerParams(
                         use_tc_tiling_on_sc=False),
                     name="sc_seg_sum_deg")(table2, src2, dst, zf, zd, ones8)


def _seg_sum(table2, src2, dst, zf):
    scratch = [
        pltpu.VMEM((EPC,), jnp.int32),
        pltpu.VMEM((EPC,), jnp.int32),
        pltpu.VMEM((CH, DH), jnp.float32),
        pltpu.VMEM((CH, DH), jnp.float32),
        pltpu.VMEM_SHARED((NP, DH), jnp.float32),
        pltpu.SemaphoreType.DMA,
        pltpu.SemaphoreType.DMA,
    ]
    out_type = jax.ShapeDtypeStruct((NC, NP, DH), jnp.float32)
    return pl.kernel(functools.partial(_seg_sum_body, False), out_type,
                     mesh=_sc_mesh(), scratch_types=scratch,
                     compiler_params=pltpu.CompilerParams(
                         use_tc_tiling_on_sc=False),
                     name="sc_seg_sum")(table2, src2, dst, zf)


EPW = E // (NC * NS)


def _edge_score_body(ac_hbm, src_hbm, dst_hbm, out_hbm,
                     ac_v, sidx_v, didx_v, out_v):
    c = lax.axis_index("c")
    s = lax.axis_index("s")
    wid = s * NC + c
    base = wid * EPW

    pltpu.sync_copy(ac_hbm, ac_v)
    pltpu.sync_copy(src_hbm.at[pl.ds(base, EPW)], sidx_v)
    pltpu.sync_copy(dst_hbm.at[pl.ds(base, EPW)], didx_v)

    def step(i, carry):
        o = i * L
        sv = sidx_v[pl.ds(o, L)]
        dv = didx_v[pl.ds(o, L)]
        a = plsc.load_gather(ac_v, [sv * 2])
        b = plsc.load_gather(ac_v, [dv * 2 + 1])
        out_v[pl.ds(o, L)] = a + b
        return carry

    lax.fori_loop(0, EPW // L, step, 0)
    pltpu.sync_copy(out_v, out_hbm.at[pl.ds(base, EPW)])


def _edge_score(ac, src, dst):
    scratch = [
        pltpu.VMEM((NP * 2,), jnp.float32),
        pltpu.VMEM((EPW,), jnp.int32),
        pltpu.VMEM((EPW,), jnp.int32),
        pltpu.VMEM((EPW,), jnp.float32),
    ]
    out_type = jax.ShapeDtypeStruct((E,), jnp.float32)
    return pl.kernel(_edge_score_body, out_type,
                     mesh=_sc_mesh(), scratch_types=scratch,
                     compiler_params=pltpu.CompilerParams(
                         needs_layout_passes=False),
                     name="sc_edge_score")(ac, src, dst)


def _conv1_body(x_b, agg_b, deg_b, w_b, b_b, out_b):
    deg = deg_b[0, :, 0:1]
    agg = jnp.concatenate([agg_b[0], agg_b[1]], axis=1)
    hn = (agg + x_b[...]) / (deg + 1.0)
    acc = jnp.dot(hn, w_b[...], preferred_element_type=jnp.float32)
    out_b[...] = jnp.maximum(acc + b_b[...], 0.0)


def _conv1(x, agg1, degp, W1, b1):
    grid = (NP // ROWS_TC,)
    return pl.pallas_call(
        _conv1_body,
        grid=grid,
        in_specs=[
            pl.BlockSpec((ROWS_TC, D), lambda i: (i, 0)),
            pl.BlockSpec((NC, ROWS_TC, DH), lambda i: (0, i, 0)),
            pl.BlockSpec((NC, ROWS_TC, DEGW), lambda i: (0, i, 0)),
            pl.BlockSpec((D, D), lambda i: (0, 0)),
            pl.BlockSpec((1, D), lambda i: (0, 0)),
        ],
        out_specs=pl.BlockSpec((ROWS_TC, D), lambda i: (i, 0)),
        out_shape=jax.ShapeDtypeStruct((NP, D), jnp.float32),
    )(x, agg1, degp, W1, b1.reshape(1, D))


def _conv2_body(h_b, agg_b, deg_b, ws_b, wn_b, b_b, wp_b, bp_b, out_b):
    deg = deg_b[0, :, 0:1]
    degc = jnp.maximum(deg, 1.0)
    agg = jnp.concatenate([agg_b[0], agg_b[1]], axis=1)
    hn = agg / degc
    h2 = (jnp.dot(h_b[...], ws_b[...], preferred_element_type=jnp.float32)
          + jnp.dot(hn, wn_b[...], preferred_element_type=jnp.float32)
          + b_b[...])
    out_b[...] = jnp.dot(h2, wp_b[...], preferred_element_type=jnp.float32) + bp_b[...]


def _conv2(h, agg2, degp, W2s, W2n, b2, Wp2, bp2):
    grid = (NP // ROWS_TC,)
    return pl.pallas_call(
        _conv2_body,
        grid=grid,
        in_specs=[
            pl.BlockSpec((ROWS_TC, D), lambda i: (i, 0)),
            pl.BlockSpec((NC, ROWS_TC, DH), lambda i: (0, i, 0)),
            pl.BlockSpec((NC, ROWS_TC, DEGW), lambda i: (0, i, 0)),
            pl.BlockSpec((D, D), lambda i: (0, 0)),
            pl.BlockSpec((D, D), lambda i: (0, 0)),
            pl.BlockSpec((1, D), lambda i: (0, 0)),
            pl.BlockSpec((D, 2), lambda i: (0, 0)),
            pl.BlockSpec((1, 2), lambda i: (0, 0)),
        ],
        out_specs=pl.BlockSpec((ROWS_TC, 2), lambda i: (i, 0)),
        out_shape=jax.ShapeDtypeStruct((NP, 2), jnp.float32),
    )(h, agg2, degp, W2s, W2n, b2.reshape(1, D), Wp2, bp2)


def kernel(x, edge_index, W1_neigh, b1, W2_self, W2_neigh, b2, W_pred, b_pred):
    src = edge_index[0]
    dst = edge_index[1]
    src2 = jnp.stack([src * 2, src * 2 + 1])
    xp = jnp.pad(x, ((0, NP - N), (0, 0)))
    zf = jnp.zeros((NP, DH), jnp.float32)
    zd = jnp.zeros((NP, DEGW), jnp.float32)
    ones8 = jnp.ones((CH, DEGW), jnp.float32)

    agg1, degp = _seg_sum_deg(
        xp.reshape(NP * 2, DH), src2, dst, zf, zd, ones8)
    h = _conv1(xp, agg1, degp, W1_neigh, b1)
    agg2 = _seg_sum(h.reshape(NP * 2, DH), src2, dst, zf)

    Wp2 = W_pred[:, 0].reshape(2, D).T
    bp2 = jnp.concatenate([b_pred, jnp.zeros((1,), jnp.float32)]).reshape(1, 2)
    ac = _conv2(h, agg2, degp, W2_self, W2_neigh, b2, Wp2, bp2)

    score = _edge_score(ac.reshape(NP * 2), src, dst)
    return score.reshape(E, 1)


# speedup vs baseline: 13.6281x; 1.2838x over previous
import functools

import jax
import jax.numpy as jnp
from jax import lax
from jax.experimental import pallas as pl
from jax.experimental.pallas import tpu as pltpu
from jax.experimental.pallas import tpu_sc as plsc

N = 10000
NP = 10240
E = 320000
D = 128
DEGW = 8

NC, NS, L = 2, 16, 16
EPC = E // NS
CH = 80
NCHUNK = EPC // CH
RPT = NP // NS

ROWS_TC = 2048

DH = D // 2


def _sc_mesh():
    return plsc.VectorSubcoreMesh(
        core_axis_name="c", subcore_axis_name="s",
        num_cores=NC, num_subcores=NS)


NB = 4


def _seg_sum_body(with_deg, *refs):
    if with_deg:
        (table_hbm, src2_hbm, dst_hbm, zf_hbm, zd_hbm, ones_hbm,
         out_hbm, deg_hbm,
         sidx_all, didx_all, rows0, rows1, rows2, rows3, ones_v,
         acc_sh, deg_sh,
         sg0, sg1, sg2, sg3, ss0, ss1, ss2, ss3, sd0, sd1, sd2, sd3) = refs
        semd = (sd0, sd1, sd2, sd3)
    else:
        (table_hbm, src2_hbm, dst_hbm, zf_hbm,
         out_hbm,
         sidx_all, didx_all, rows0, rows1, rows2, rows3,
         acc_sh,
         sg0, sg1, sg2, sg3, ss0, ss1, ss2, ss3) = refs

    c = lax.axis_index("c")
    s = lax.axis_index("s")
    r0 = s * RPT
    base = s * EPC
    rows = (rows0, rows1, rows2, rows3)
    semg = (sg0, sg1, sg2, sg3)
    sems = (ss0, ss1, ss2, ss3)

    pltpu.sync_copy(zf_hbm.at[pl.ds(r0, RPT)], acc_sh.at[pl.ds(r0, RPT)])
    if with_deg:
        pltpu.sync_copy(zd_hbm.at[pl.ds(r0, RPT)], deg_sh.at[pl.ds(r0, RPT)])
        pltpu.sync_copy(ones_hbm, ones_v)

    pltpu.sync_copy(src2_hbm.at[c, pl.ds(base, EPC)], sidx_all)
    pltpu.sync_copy(dst_hbm.at[pl.ds(base, EPC)], didx_all)
    plsc.subcore_barrier()

    def fetch(k, b):
        pltpu.async_copy(table_hbm.at[sidx_all.at[pl.ds(k * CH, CH)]],
                         rows[b], semg[b])

    def scat(k, b):
        didx = didx_all.at[pl.ds(k * CH, CH)]
        pltpu.make_async_copy(table_hbm.at[sidx_all.at[pl.ds(k * CH, CH)]],
                              rows[b], semg[b]).wait()
        pltpu.async_copy(rows[b], acc_sh.at[didx], sems[b], add=True)
        if with_deg:
            pltpu.async_copy(ones_v, deg_sh.at[didx], semd[b], add=True)

    def wait_scat(k, b):
        didx = didx_all.at[pl.ds(k * CH, CH)]
        pltpu.make_async_copy(rows[b], acc_sh.at[didx], sems[b]).wait()
        if with_deg:
            pltpu.make_async_copy(ones_v, deg_sh.at[didx], semd[b]).wait()

    for b in range(NB):
        fetch(b, b)

    def step(j, carry):
        k = j * NB
        for b in range(NB):
            scat(k + b, b)
        for b in range(NB):
            wait_scat(k + b, b)
            fetch(k + NB + b, b)
        return carry

    lax.fori_loop(0, NCHUNK // NB - 1, step, 0)
    kk = NCHUNK - NCHUNK % NB - NB
    for b in range(NB):
        scat(kk + b, b)
    for b in range(NCHUNK % NB):
        wait_scat(kk + b, b)
        fetch(kk + NB + b, b)
    for b in range(NCHUNK % NB):
        scat(kk + NB + b, b)
    for b in range(NCHUNK % NB, NB):
        wait_scat(kk + b, b)
    for b in range(NCHUNK % NB):
        wait_scat(kk + NB + b, b)
    plsc.subcore_barrier()

    pltpu.sync_copy(acc_sh.at[pl.ds(r0, RPT)], out_hbm.at[c, pl.ds(r0, RPT)])
    if with_deg:
        pltpu.sync_copy(deg_sh.at[pl.ds(r0, RPT)],
                        deg_hbm.at[c, pl.ds(r0, RPT)])


def _seg_sum_deg(table2, src2, dst, zf, zd, ones8):
    scratch = [
        pltpu.VMEM((EPC,), jnp.int32),
        pltpu.VMEM((EPC,), jnp.int32),
        pltpu.VMEM((CH, DH), jnp.float32),
        pltpu.VMEM((CH, DH), jnp.float32),
        pltpu.VMEM((CH, DH), jnp.float32),
        pltpu.VMEM((CH, DH), jnp.float32),
        pltpu.VMEM((CH, DEGW), jnp.float32),
        pltpu.VMEM_SHARED((NP, DH), jnp.float32),
        pltpu.VMEM_SHARED((NP, DEGW), jnp.float32),
    ] + [pltpu.SemaphoreType.DMA] * 12
    out_type = (jax.ShapeDtypeStruct((NC, NP, DH), jnp.float32),
                jax.ShapeDtypeStruct((NC, NP, DEGW), jnp.float32))
    return pl.kernel(functools.partial(_seg_sum_body, True), out_type,
                     mesh=_sc_mesh(), scratch_types=scratch,
                     compiler_params=pltpu.CompilerParams(
                         use_tc_tiling_on_sc=False),
                     name="sc_seg_sum_deg")(table2, src2, dst, zf, zd, ones8)


def _seg_sum(table2, src2, dst, zf):
    scratch = [
        pltpu.VMEM((EPC,), jnp.int32),
        pltpu.VMEM((EPC,), jnp.int32),
        pltpu.VMEM((CH, DH), jnp.float32),
        pltpu.VMEM((CH, DH), jnp.float32),
        pltpu.VMEM((CH, DH), jnp.float32),
        pltpu.VMEM((CH, DH), jnp.float32),
        pltpu.VMEM_SHARED((NP, DH), jnp.float32),
    ] + [pltpu.SemaphoreType.DMA] * 8
    out_type = jax.ShapeDtypeStruct((NC, NP, DH), jnp.float32)
    return pl.kernel(functools.partial(_seg_sum_body, False), out_type,
                     mesh=_sc_mesh(), scratch_types=scratch,
                     compiler_params=pltpu.CompilerParams(
                         use_tc_tiling_on_sc=False),
                     name="sc_seg_sum")(table2, src2, dst, zf)


EPW = E // (NC * NS)


def _edge_score_body(ac_hbm, src_hbm, dst_hbm, out_hbm,
                     ac_v, sidx_v, didx_v, out_v):
    c = lax.axis_index("c")
    s = lax.axis_index("s")
    wid = s * NC + c
    base = wid * EPW

    pltpu.sync_copy(ac_hbm, ac_v)
    pltpu.sync_copy(src_hbm.at[pl.ds(base, EPW)], sidx_v)
    pltpu.sync_copy(dst_hbm.at[pl.ds(base, EPW)], didx_v)

    def step(i, carry):
        o = i * L
        sv = sidx_v[pl.ds(o, L)]
        dv = didx_v[pl.ds(o, L)]
        a = plsc.load_gather(ac_v, [sv * 2])
        b = plsc.load_gather(ac_v, [dv * 2 + 1])
        out_v[pl.ds(o, L)] = a + b
        return carry

    lax.fori_loop(0, EPW // L, step, 0)
    pltpu.sync_copy(out_v, out_hbm.at[pl.ds(base, EPW)])


def _edge_score(ac, src, dst):
    scratch = [
        pltpu.VMEM((NP * 2,), jnp.float32),
        pltpu.VMEM((EPW,), jnp.int32),
        pltpu.VMEM((EPW,), jnp.int32),
        pltpu.VMEM((EPW,), jnp.float32),
    ]
    out_type = jax.ShapeDtypeStruct((E,), jnp.float32)
    return pl.kernel(_edge_score_body, out_type,
                     mesh=_sc_mesh(), scratch_types=scratch,
                     compiler_params=pltpu.CompilerParams(
                         needs_layout_passes=False),
                     name="sc_edge_score")(ac, src, dst)


def _conv1_body(x_b, agg_b, deg_b, w_b, b_b, out_b):
    deg = deg_b[0, :, 0:1]
    agg = jnp.concatenate([agg_b[0], agg_b[1]], axis=1)
    hn = (agg + x_b[...]) / (deg + 1.0)
    acc = jnp.dot(hn, w_b[...], preferred_element_type=jnp.float32)
    out_b[...] = jnp.maximum(acc + b_b[...], 0.0)


def _conv1(x, agg1, degp, W1, b1):
    grid = (NP // ROWS_TC,)
    return pl.pallas_call(
        _conv1_body,
        grid=grid,
        in_specs=[
            pl.BlockSpec((ROWS_TC, D), lambda i: (i, 0)),
            pl.BlockSpec((NC, ROWS_TC, DH), lambda i: (0, i, 0)),
            pl.BlockSpec((NC, ROWS_TC, DEGW), lambda i: (0, i, 0)),
            pl.BlockSpec((D, D), lambda i: (0, 0)),
            pl.BlockSpec((1, D), lambda i: (0, 0)),
        ],
        out_specs=pl.BlockSpec((ROWS_TC, D), lambda i: (i, 0)),
        out_shape=jax.ShapeDtypeStruct((NP, D), jnp.float32),
    )(x, agg1, degp, W1, b1.reshape(1, D))


def _conv2_body(h_b, agg_b, deg_b, ws_b, wn_b, b_b, wp_b, bp_b, out_b):
    deg = deg_b[0, :, 0:1]
    degc = jnp.maximum(deg, 1.0)
    agg = jnp.concatenate([agg_b[0], agg_b[1]], axis=1)
    hn = agg / degc
    h2 = (jnp.dot(h_b[...], ws_b[...], preferred_element_type=jnp.float32)
          + jnp.dot(hn, wn_b[...], preferred_element_type=jnp.float32)
          + b_b[...])
    out_b[...] = jnp.dot(h2, wp_b[...], preferred_element_type=jnp.float32) + bp_b[...]


def _conv2(h, agg2, degp, W2s, W2n, b2, Wp2, bp2):
    grid = (NP // ROWS_TC,)
    return pl.pallas_call(
        _conv2_body,
        grid=grid,
        in_specs=[
            pl.BlockSpec((ROWS_TC, D), lambda i: (i, 0)),
            pl.BlockSpec((NC, ROWS_TC, DH), lambda i: (0, i, 0)),
            pl.BlockSpec((NC, ROWS_TC, DEGW), lambda i: (0, i, 0)),
            pl.BlockSpec((D, D), lambda i: (0, 0)),
            pl.BlockSpec((D, D), lambda i: (0, 0)),
            pl.BlockSpec((1, D), lambda i: (0, 0)),
            pl.BlockSpec((D, 2), lambda i: (0, 0)),
            pl.BlockSpec((1, 2), lambda i: (0, 0)),
        ],
        out_specs=pl.BlockSpec((ROWS_TC, 2), lambda i: (i, 0)),
        out_shape=jax.ShapeDtypeStruct((NP, 2), jnp.float32),
    )(h, agg2, degp, W2s, W2n, b2.reshape(1, D), Wp2, bp2)


def kernel(x, edge_index, W1_neigh, b1, W2_self, W2_neigh, b2, W_pred, b_pred):
    src = edge_index[0]
    dst = edge_index[1]
    src2 = jnp.stack([src * 2, src * 2 + 1])
    xp = jnp.pad(x, ((0, NP - N), (0, 0)))
    zf = jnp.zeros((NP, DH), jnp.float32)
    zd = jnp.zeros((NP, DEGW), jnp.float32)
    ones8 = jnp.ones((CH, DEGW), jnp.float32)

    agg1, degp = _seg_sum_deg(
        xp.reshape(NP * 2, DH), src2, dst, zf, zd, ones8)
    h = _conv1(xp, agg1, degp, W1_neigh, b1)
    agg2 = _seg_sum(h.reshape(NP * 2, DH), src2, dst, zf)

    Wp2 = W_pred[:, 0].reshape(2, D).T
    bp2 = jnp.concatenate([b_pred, jnp.zeros((1,), jnp.float32)]).reshape(1, 2)
    ac = _conv2(h, agg2, degp, W2_self, W2_neigh, b2, Wp2, bp2)

    score = _edge_score(ac.reshape(NP * 2), src, dst)
    return score.reshape(E, 1)
